# baseline (speedup 1.0000x reference)
import functools

import jax
import jax.numpy as jnp
import numpy as np
from jax import lax
from jax.experimental import pallas as pl
from jax.experimental.pallas import tpu as pltpu
from jax.experimental.pallas import tpu_sc as plsc

N = 10000
E = 320000
H = 128
K = 30
D_LAT = 3 * H + 1

NP = 10240
NDUM = 16
E_PAD = 323584
CH = 128
EPT_AGG = E_PAD // 32
NCH_AGG = EPT_AGG // CH
EPT_DEG = E_PAD // 16
NCH_DEG = EPT_DEG // CH
RPT = NP // 16
F32 = jnp.float32
I32 = jnp.int32
NEG_INF = float(np.finfo(np.float32).min)


def _zeros16():
    return jnp.zeros((16,), F32)


def _sc_deg_norm(srcp, dstp):
    mesh = plsc.VectorSubcoreMesh(core_axis_name="c", subcore_axis_name="s")

    @functools.partial(
        pl.kernel,
        out_type=(
            jax.ShapeDtypeStruct((NP, H), F32),
            jax.ShapeDtypeStruct((NP, H), F32),
        ),
        mesh=mesh,
        scratch_types=[
            pltpu.VMEM((CH,), I32),
            pltpu.VMEM((CH,), F32),
            pltpu.VMEM((320,), F32),
            pltpu.VMEM((320, H), F32),
            pltpu.VMEM_SHARED((NP,), F32),
        ],
    )
    def k(src_h, dst_h, onb_h, inb_h, idxb, ones_v, degb, rowb, acc):
        c = lax.axis_index("c")
        s = lax.axis_index("s")
        for q in range(CH // 16):
            ones_v[pl.ds(q * 16, 16)] = jnp.ones((16,), F32)
        for q in range(20):
            degb[pl.ds(q * 16, 16)] = _zeros16()
        pltpu.sync_copy(degb, acc.at[pl.ds(s * RPT, 320)])
        pltpu.sync_copy(degb, acc.at[pl.ds(s * RPT + 320, 320)])
        plsc.subcore_barrier()

        def run(idx_hbm, out_hbm):
            def body(ci, _):
                off = s * EPT_DEG + ci * CH
                pltpu.sync_copy(idx_hbm.at[pl.ds(off, CH)], idxb)
                pltpu.sync_copy(ones_v, acc.at[idxb], add=True)
                return 0

            lax.fori_loop(0, NCH_DEG, body, 0)
            plsc.subcore_barrier()
            for half in range(2):
                r0 = s * RPT + half * 320
                pltpu.sync_copy(acc.at[pl.ds(r0, 320)], degb)

                def rowbody(g, _):
                    v = degb[pl.ds(g * 16, 16)]
                    for l in range(16):
                        row = jnp.full((16,), v[l], F32)
                        for q in range(H // 16):
                            rowb[g * 16 + l, pl.ds(q * 16, 16)] = row
                    return 0

                lax.fori_loop(0, 20, rowbody, 0)
                pltpu.sync_copy(rowb, out_hbm.at[pl.ds(r0, 320)])

        @pl.when(c == 0)
        def _():
            run(src_h, onb_h)

        @pl.when(c == 1)
        def _():
            run(dst_h, inb_h)

    return k(srcp, dstp)


def _sc_embed(z_table, z_pad):
    mesh = plsc.VectorSubcoreMesh(core_axis_name="c", subcore_axis_name="s")

    @functools.partial(
        pl.kernel,
        out_type=jax.ShapeDtypeStruct((NP, H), F32),
        mesh=mesh,
        scratch_types=[
            pltpu.VMEM((64,), I32),
            pltpu.VMEM((64, H), F32),
            pltpu.SemaphoreType.DMA,
        ],
    )
    def k(tab_h, z_h, out_h, idxb, rows, sem):
        c = lax.axis_index("c")
        s = lax.axis_index("s")
        wid = s * 2 + c
        base = wid * 320
        for kk in range(5):
            off = base + kk * 64
            pltpu.sync_copy(z_h.at[pl.ds(off, 64)], idxb)
            pltpu.async_copy(tab_h.at[idxb], rows, sem).wait()
            pltpu.sync_copy(rows, out_h.at[pl.ds(off, 64)])

    return k(z_table, z_pad)


def _sc_agg(t, srcp, dstp):
    mesh = plsc.VectorSubcoreMesh(core_axis_name="c", subcore_axis_name="s")

    @functools.partial(
        pl.kernel,
        out_type=jax.ShapeDtypeStruct((2, NP, H), F32),
        mesh=mesh,
        scratch_types=[
            pltpu.VMEM((CH,), I32),
            pltpu.VMEM((CH,), I32),
            pltpu.VMEM((CH, H), F32),
            pltpu.VMEM_SHARED((NP, H), F32),
            pltpu.SemaphoreType.DMA,
        ],
    )
    def k(t_h, src_h, dst_h, out_h, sidx, didx, rbuf, acc, sem):
        c = lax.axis_index("c")
        s = lax.axis_index("s")
        wid = s * 2 + c

        def zbody(r, _):
            for q in range(H // 16):
                rbuf[r, pl.ds(q * 16, 16)] = _zeros16()
            return 0

        lax.fori_loop(0, CH, zbody, 0)
        for b in range(RPT // CH):
            pltpu.sync_copy(rbuf, acc.at[pl.ds(s * RPT + b * CH, CH)])
        plsc.subcore_barrier()

        def body(ci, _):
            off = wid * EPT_AGG + ci * CH
            pltpu.sync_copy(src_h.at[pl.ds(off, CH)], sidx)
            pltpu.async_copy(t_h.at[sidx], rbuf, sem).wait()
            pltpu.sync_copy(dst_h.at[pl.ds(off, CH)], didx)
            pltpu.sync_copy(rbuf, acc.at[didx], add=True)
            return 0

        lax.fori_loop(0, NCH_AGG, body, 0)
        plsc.subcore_barrier()
        pltpu.sync_copy(acc.at[pl.ds(s * RPT, RPT)],
                        out_h.at[c, pl.ds(s * RPT, RPT)])

    return k(t, srcp, dstp)


RB = 1280


def _tc_first(h0, onb, wg):
    def body(h_ref, o_ref, w_ref, t_ref):
        onorm = jnp.power(jnp.maximum(o_ref[...], 1.0), -0.5)
        t_ref[...] = lax.dot_general(
            h_ref[...] * onorm, w_ref[...],
            (((1,), (0,)), ((), ())), preferred_element_type=F32)

    return pl.pallas_call(
        body,
        grid=(NP // RB,),
        in_specs=[
            pl.BlockSpec((RB, H), lambda i: (i, 0)),
            pl.BlockSpec((RB, H), lambda i: (i, 0)),
            pl.BlockSpec((H, H), lambda i: (0, 0)),
        ],
        out_specs=pl.BlockSpec((RB, H), lambda i: (i, 0)),
        out_shape=jax.ShapeDtypeStruct((NP, H), F32),
    )(h0, onb, wg)


def _tc_combine(part, inb, onb, bg, wf, bf, wgn):
    def body(p_ref, i_ref, o_ref, bg_ref, wf_ref, bf_ref, wg_ref,
             h_ref, t_ref):
        inorm = jnp.power(jnp.maximum(i_ref[...], 1.0), -0.5)
        onorm = jnp.power(jnp.maximum(o_ref[...], 1.0), -0.5)
        g = (p_ref[0] + p_ref[1]) * inorm + bg_ref[...]
        a = lax.dot_general(jnp.maximum(g, 0.0), wf_ref[...],
                            (((1,), (0,)), ((), ())),
                            preferred_element_type=F32) + bf_ref[...]
        h = jnp.tanh(a)
        h_ref[...] = h
        t_ref[...] = lax.dot_general(h * onorm, wg_ref[...],
                                     (((1,), (0,)), ((), ())),
                                     preferred_element_type=F32)

    return pl.pallas_call(
        body,
        grid=(NP // RB,),
        in_specs=[
            pl.BlockSpec((2, RB, H), lambda i: (0, i, 0)),
            pl.BlockSpec((RB, H), lambda i: (i, 0)),
            pl.BlockSpec((RB, H), lambda i: (i, 0)),
            pl.BlockSpec((1, H), lambda i: (0, 0)),
            pl.BlockSpec((H, H), lambda i: (0, 0)),
            pl.BlockSpec((1, H), lambda i: (0, 0)),
            pl.BlockSpec((H, H), lambda i: (0, 0)),
        ],
        out_specs=[pl.BlockSpec((RB, H), lambda i: (i, 0)),
                   pl.BlockSpec((RB, H), lambda i: (i, 0))],
        out_shape=[jax.ShapeDtypeStruct((NP, H), F32),
                   jax.ShapeDtypeStruct((NP, H), F32)],
    )(part, inb, onb, bg, wf, bf, wgn)


def _bitonic_lanes(X):
    R, n = X.shape
    li = lax.broadcasted_iota(I32, (R, n), 1)
    k = 2
    while k <= n:
        j = k // 2
        while j >= 1:
            bitj0 = (li & j) == 0
            asc = (li & k) == 0
            keepmin = bitj0 == asc
            left = jnp.concatenate([X[:, j:], X[:, :j]], axis=1)
            right = jnp.concatenate([X[:, n - j:], X[:, :n - j]], axis=1)
            xp = jnp.where(bitj0, left, right)
            X = jnp.where(keepmin, jnp.minimum(X, xp), jnp.maximum(X, xp))
            j //= 2
        k *= 2
    return X


def _tc_pre4(part4, inb, bg4, wf4, bf4):
    def body(p_ref, i_ref, bg_ref, wf_ref, bf_ref, a4_ref):
        inorm = jnp.power(jnp.maximum(i_ref[...], 1.0), -0.5)
        g4 = (p_ref[0] + p_ref[1]) * inorm + bg_ref[...]
        a4 = jnp.sum(jnp.maximum(g4, 0.0) * wf_ref[...], axis=1,
                     keepdims=True) + bf_ref[...]
        a4_ref[...] = jnp.broadcast_to(jnp.tanh(a4), (RB, H))

    blk = pl.BlockSpec((RB, H), lambda i: (i, 0))
    cst = lambda shape: pl.BlockSpec(shape, lambda i: tuple(0 for _ in shape))
    return pl.pallas_call(
        body,
        grid=(NP // RB,),
        in_specs=[
            pl.BlockSpec((2, RB, H), lambda i: (0, i, 0)),
            blk,
            cst((1, H)), cst((1, H)), cst((1, 1)),
        ],
        out_specs=blk,
        out_shape=jax.ShapeDtypeStruct((NP, H), F32),
    )(part4, inb, bg4, wf4, bf4)


def _tc_rowmax(h1, h2, h3, h4b):
    def body(h1_ref, h2_ref, h3_ref, h4_ref, m_ref):
        i = pl.program_id(0)
        m = jnp.maximum(
            jnp.maximum(jnp.max(h1_ref[...], axis=1, keepdims=True),
                        jnp.max(h2_ref[...], axis=1, keepdims=True)),
            jnp.maximum(jnp.max(h3_ref[...], axis=1, keepdims=True),
                        h4_ref[:, 0:1]))
        rid = i * RB + lax.broadcasted_iota(I32, (RB, 1), 0)
        m = jnp.where(rid < N, m, NEG_INF)
        m_ref[...] = jnp.broadcast_to(m, (RB, H))

    blk = pl.BlockSpec((RB, H), lambda i: (i, 0))
    return pl.pallas_call(
        body,
        grid=(NP // RB,),
        in_specs=[blk, blk, blk, blk],
        out_specs=blk,
        out_shape=jax.ShapeDtypeStruct((NP, H), F32),
    )(h1, h2, h3, h4b)


def _tc_topk(mb):
    def body(m_ref, oh_ref):
        m = m_ref[:, 0:1]
        rid = lax.broadcasted_iota(I32, (NP, 1), 0)
        row_iota = lax.broadcasted_iota(I32, (1, NP), 1)
        sel_iota = lax.broadcasted_iota(I32, (32, 1), 0)

        def topk_body(j, carry):
            mc, oh = carry
            vmax = jnp.max(mc)
            idx = jnp.min(jnp.where(mc == vmax, rid, NP))
            oh = oh + ((sel_iota == j).astype(F32) *
                       (row_iota == idx).astype(F32))
            mc = jnp.where(rid == idx, NEG_INF, mc)
            return (mc, oh)

        _, oh = lax.fori_loop(0, K, topk_body,
                              (m, jnp.zeros((32, NP), F32)))
        oh_ref[...] = oh

    return pl.pallas_call(
        body,
        out_shape=jax.ShapeDtypeStruct((32, NP), F32),
    )(mb)


def _tc_gather(oh, h1, h2, h3, h4b):
    def body(oh_ref, h1_ref, h2_ref, h3_ref, h4_ref, out_ref):
        i = pl.program_id(0)
        dn = (((1,), (0,)), ((), ()))
        hp = dict(preferred_element_type=F32,
                  precision=lax.Precision.HIGHEST)
        ohb = oh_ref[...]
        p1 = lax.dot_general(ohb, h1_ref[...], dn, **hp)
        p2 = lax.dot_general(ohb, h2_ref[...], dn, **hp)
        p3 = lax.dot_general(ohb, h3_ref[...], dn, **hp)
        p4 = lax.dot_general(ohb, h4_ref[:, 0:1], dn, **hp)
        part = jnp.concatenate([p1, p2, p3, p4, jnp.zeros((32, 127), F32)],
                               axis=1)

        @pl.when(i == 0)
        def _():
            out_ref[...] = jnp.zeros((32, 512), F32)

        out_ref[...] += part

    blk = pl.BlockSpec((RB, H), lambda i: (i, 0))
    return pl.pallas_call(
        body,
        grid=(NP // RB,),
        in_specs=[pl.BlockSpec((32, RB), lambda i: (0, i)),
                  blk, blk, blk, blk],
        out_specs=pl.BlockSpec((32, 512), lambda i: (0, 0)),
        out_shape=jax.ShapeDtypeStruct((32, 512), F32),
        compiler_params=pltpu.CompilerParams(
            dimension_semantics=("arbitrary",)),
    )(oh, h1, h2, h3, h4b)


def _tc_readout(pooled, w1r, b1, se, so, st, w2r, b2, l1r, l1b, l2w, l2b):
    def body(pool_ref, w1_ref, b1_ref, se_ref, so_ref, st_ref, w2_ref,
             b2_ref, l1_ref, l1b_ref, l2_ref, l2b_ref, out_ref):
        lane = lax.broadcasted_iota(I32, (32, 512), 1)
        x32 = jnp.where(lane < D_LAT, pool_ref[...], jnp.inf)
        xs = _bitonic_lanes(x32)[0:K, 0:D_LAT]
        y1 = jnp.maximum(
            lax.dot_general(xs, w1_ref[...], (((1,), (0,)), ((), ())),
                            preferred_element_type=F32) + b1_ref[...], 0.0)
        hp = dict(preferred_element_type=F32,
                  precision=lax.Precision.HIGHEST)
        z = jnp.maximum(
            lax.dot_general(se_ref[...], y1, (((1,), (0,)), ((), ())), **hp),
            lax.dot_general(so_ref[...], y1, (((1,), (0,)), ((), ())), **hp))
        wins = jnp.concatenate(
            [lax.dot_general(st_ref[t], z, (((1,), (0,)), ((), ())), **hp)
             for t in range(5)],
            axis=1)
        y2 = jnp.maximum(
            lax.dot_general(wins, w2_ref[...], (((1,), (0,)), ((), ())),
                            preferred_element_type=F32) + b2_ref[...], 0.0)
        accum = l1b_ref[...]
        for p in range(11):
            accum = accum + lax.dot_general(
                y2[p:p + 1, :], l1_ref[p], (((1,), (0,)), ((), ())),
                preferred_element_type=F32)
        x = jnp.maximum(accum, 0.0)
        out_ref[...] = jnp.sum(x * l2_ref[...], axis=1,
                               keepdims=True) + l2b_ref[...]

    return pl.pallas_call(
        body,
        out_shape=jax.ShapeDtypeStruct((1, 1), F32),
    )(pooled, w1r, b1, se, so, st, w2r, b2, l1r, l1b, l2w, l2b)


def kernel(params, z, edge_index):
    src = edge_index[0].astype(I32)
    dst = edge_index[1].astype(I32)
    pad = N + (jnp.arange(E_PAD - E, dtype=I32) % NDUM)
    srcp = jnp.concatenate([src, pad])
    dstp = jnp.concatenate([dst, pad])
    z_pad = jnp.concatenate([z.astype(I32), jnp.zeros((NP - N,), I32)])

    convs = params['convs']
    onb, inb = _sc_deg_norm(srcp, dstp)
    h0 = _sc_embed(params['z_table'], z_pad)

    t = _tc_first(h0, onb, convs[0]['Wg'])
    hs = []
    for i in range(3):
        part = _sc_agg(t, srcp, dstp)
        h, t = _tc_combine(part, inb, onb,
                           convs[i]['bg'].reshape(1, H),
                           convs[i]['Wf'],
                           convs[i]['bf'].reshape(1, H),
                           convs[i + 1]['Wg'])
        hs.append(h)
    part4 = _sc_agg(t, srcp, dstp)
    h4b = _tc_pre4(part4, inb,
                   convs[3]['bg'].reshape(1, H),
                   convs[3]['Wf'].reshape(1, H),
                   convs[3]['bf'].reshape(1, 1))
    mb = _tc_rowmax(hs[0], hs[1], hs[2], h4b)

    se = np.zeros((15, K), np.float32)
    so = np.zeros((15, K), np.float32)
    for kk in range(15):
        se[kk, 2 * kk] = 1.0
        so[kk, 2 * kk + 1] = 1.0
    st = np.zeros((5, 11, 15), np.float32)
    for t_ in range(5):
        for p_ in range(11):
            st[t_, p_, p_ + t_] = 1.0

    w1r = params['conv1_W'][:, 0, :].T
    w2r = jnp.transpose(params['conv2_W'], (2, 1, 0)).reshape(80, 32)
    l1r = params['lin1_W'].reshape(32, 11, H).transpose(1, 0, 2)

    oh = _tc_topk(mb)
    pooled = _tc_gather(oh, hs[0], hs[1], hs[2], h4b)
    out = _tc_readout(
        pooled,
        w1r,
        params['conv1_b'].reshape(1, 16),
        jnp.asarray(se), jnp.asarray(so), jnp.asarray(st),
        w2r,
        params['conv2_b'].reshape(1, 32),
        l1r,
        params['lin1_b'].reshape(1, H),
        params['lin2_W'].reshape(1, H),
        params['lin2_b'].reshape(1, 1),
    )
    return out


# speedup vs baseline: 1.1708x; 1.1708x over previous
import functools

import jax
import jax.numpy as jnp
import numpy as np
from jax import lax
from jax.experimental import pallas as pl
from jax.experimental.pallas import tpu as pltpu
from jax.experimental.pallas import tpu_sc as plsc

N = 10000
E = 320000
H = 128
K = 30
D_LAT = 3 * H + 1

NP = 10240
NDUM = 16
E_PAD = 323584
CH = 128
EPT_AGG = E_PAD // 32
NCH_AGG = EPT_AGG // CH
EPT_DEG = E_PAD // 16
NCH_DEG = EPT_DEG // CH
RPT = NP // 16
F32 = jnp.float32
I32 = jnp.int32
NEG_INF = float(np.finfo(np.float32).min)


def _zeros16():
    return jnp.zeros((16,), F32)


def _sc_deg_norm(srcp, dstp):
    mesh = plsc.VectorSubcoreMesh(core_axis_name="c", subcore_axis_name="s")

    @functools.partial(
        pl.kernel,
        out_type=(
            jax.ShapeDtypeStruct((NP, H), F32),
            jax.ShapeDtypeStruct((NP, H), F32),
        ),
        mesh=mesh,
        scratch_types=[
            pltpu.VMEM((CH,), I32),
            pltpu.VMEM((CH,), F32),
            pltpu.VMEM((320,), F32),
            pltpu.VMEM((320, H), F32),
            pltpu.VMEM_SHARED((NP,), F32),
        ],
    )
    def k(src_h, dst_h, onb_h, inb_h, idxb, ones_v, degb, rowb, acc):
        c = lax.axis_index("c")
        s = lax.axis_index("s")
        for q in range(CH // 16):
            ones_v[pl.ds(q * 16, 16)] = jnp.ones((16,), F32)
        for q in range(20):
            degb[pl.ds(q * 16, 16)] = _zeros16()
        pltpu.sync_copy(degb, acc.at[pl.ds(s * RPT, 320)])
        pltpu.sync_copy(degb, acc.at[pl.ds(s * RPT + 320, 320)])
        plsc.subcore_barrier()

        def run(idx_hbm, out_hbm):
            def body(ci, _):
                off = s * EPT_DEG + ci * CH
                pltpu.sync_copy(idx_hbm.at[pl.ds(off, CH)], idxb)
                pltpu.sync_copy(ones_v, acc.at[idxb], add=True)
                return 0

            lax.fori_loop(0, NCH_DEG, body, 0)
            plsc.subcore_barrier()
            for half in range(2):
                r0 = s * RPT + half * 320
                pltpu.sync_copy(acc.at[pl.ds(r0, 320)], degb)

                def rowbody(g, _):
                    v = degb[pl.ds(g * 16, 16)]
                    for l in range(16):
                        row = jnp.full((16,), v[l], F32)
                        for q in range(H // 16):
                            rowb[g * 16 + l, pl.ds(q * 16, 16)] = row
                    return 0

                lax.fori_loop(0, 20, rowbody, 0)
                pltpu.sync_copy(rowb, out_hbm.at[pl.ds(r0, 320)])

        @pl.when(c == 0)
        def _():
            run(src_h, onb_h)

        @pl.when(c == 1)
        def _():
            run(dst_h, inb_h)

    return k(srcp, dstp)


def _sc_embed(z_table, z_pad):
    mesh = plsc.VectorSubcoreMesh(core_axis_name="c", subcore_axis_name="s")

    @functools.partial(
        pl.kernel,
        out_type=jax.ShapeDtypeStruct((NP, H), F32),
        mesh=mesh,
        scratch_types=[
            pltpu.VMEM((64,), I32),
            pltpu.VMEM((64, H), F32),
            pltpu.SemaphoreType.DMA,
        ],
    )
    def k(tab_h, z_h, out_h, idxb, rows, sem):
        c = lax.axis_index("c")
        s = lax.axis_index("s")
        wid = s * 2 + c
        base = wid * 320
        for kk in range(5):
            off = base + kk * 64
            pltpu.sync_copy(z_h.at[pl.ds(off, 64)], idxb)
            pltpu.async_copy(tab_h.at[idxb], rows, sem).wait()
            pltpu.sync_copy(rows, out_h.at[pl.ds(off, 64)])

    return k(z_table, z_pad)


def _sc_agg(t, srcp, dstp):
    mesh = plsc.VectorSubcoreMesh(core_axis_name="c", subcore_axis_name="s")

    @functools.partial(
        pl.kernel,
        out_type=jax.ShapeDtypeStruct((2, NP, H), F32),
        mesh=mesh,
        scratch_types=[
            pltpu.VMEM((2, CH), I32),
            pltpu.VMEM((2, CH), I32),
            pltpu.VMEM((2, CH, H), F32),
            pltpu.VMEM_SHARED((NP, H), F32),
            pltpu.SemaphoreType.DMA,
            pltpu.SemaphoreType.DMA,
        ],
    )
    def k(t_h, src_h, dst_h, out_h, sidx, didx, rbuf, acc, gsem, ssem):
        c = lax.axis_index("c")
        s = lax.axis_index("s")
        wid = s * 2 + c
        ebase = wid * EPT_AGG

        def zbody(r, _):
            for q in range(H // 16):
                rbuf[0, r, pl.ds(q * 16, 16)] = _zeros16()
            return 0

        lax.fori_loop(0, CH, zbody, 0)
        for b in range(RPT // CH):
            pltpu.sync_copy(rbuf.at[0], acc.at[pl.ds(s * RPT + b * CH, CH)])
        plsc.subcore_barrier()

        def fetch(ci, b):
            off = ebase + ci * CH
            pltpu.sync_copy(src_h.at[pl.ds(off, CH)], sidx.at[b])
            pltpu.sync_copy(dst_h.at[pl.ds(off, CH)], didx.at[b])
            pltpu.async_copy(t_h.at[sidx.at[b]], rbuf.at[b], gsem)

        def gwait(b):
            pltpu.make_async_copy(t_h.at[sidx.at[b]], rbuf.at[b],
                                  gsem).wait()

        def scat(b):
            pltpu.async_copy(rbuf.at[b], acc.at[didx.at[b]], ssem, add=True)

        def swaitb(b):
            pltpu.make_async_copy(rbuf.at[b], acc.at[didx.at[b]],
                                  ssem).wait()

        fetch(0, 0)

        def body(g, _):
            gwait(0)
            fetch(2 * g + 1, 1)
            scat(0)
            gwait(1)
            swaitb(0)
            fetch(2 * g + 2, 0)
            scat(1)
            swaitb(1)
            return 0

        lax.fori_loop(0, (NCH_AGG - 1) // 2, body, 0)
        gwait(0)
        scat(0)
        swaitb(0)
        plsc.subcore_barrier()
        pltpu.sync_copy(acc.at[pl.ds(s * RPT, RPT)],
                        out_h.at[c, pl.ds(s * RPT, RPT)])

    return k(t, srcp, dstp)


RB = 1280


def _tc_first(h0, onb, wg):
    def body(h_ref, o_ref, w_ref, t_ref):
        onorm = jnp.power(jnp.maximum(o_ref[...], 1.0), -0.5)
        t_ref[...] = lax.dot_general(
            h_ref[...] * onorm, w_ref[...],
            (((1,), (0,)), ((), ())), preferred_element_type=F32)

    return pl.pallas_call(
        body,
        grid=(NP // RB,),
        in_specs=[
            pl.BlockSpec((RB, H), lambda i: (i, 0)),
            pl.BlockSpec((RB, H), lambda i: (i, 0)),
            pl.BlockSpec((H, H), lambda i: (0, 0)),
        ],
        out_specs=pl.BlockSpec((RB, H), lambda i: (i, 0)),
        out_shape=jax.ShapeDtypeStruct((NP, H), F32),
    )(h0, onb, wg)


def _tc_combine(part, inb, onb, bg, wf, bf, wgn):
    def body(p_ref, i_ref, o_ref, bg_ref, wf_ref, bf_ref, wg_ref,
             h_ref, t_ref):
        inorm = jnp.power(jnp.maximum(i_ref[...], 1.0), -0.5)
        onorm = jnp.power(jnp.maximum(o_ref[...], 1.0), -0.5)
        g = (p_ref[0] + p_ref[1]) * inorm + bg_ref[...]
        a = lax.dot_general(jnp.maximum(g, 0.0), wf_ref[...],
                            (((1,), (0,)), ((), ())),
                            preferred_element_type=F32) + bf_ref[...]
        h = jnp.tanh(a)
        h_ref[...] = h
        t_ref[...] = lax.dot_general(h * onorm, wg_ref[...],
                                     (((1,), (0,)), ((), ())),
                                     preferred_element_type=F32)

    return pl.pallas_call(
        body,
        grid=(NP // RB,),
        in_specs=[
            pl.BlockSpec((2, RB, H), lambda i: (0, i, 0)),
            pl.BlockSpec((RB, H), lambda i: (i, 0)),
            pl.BlockSpec((RB, H), lambda i: (i, 0)),
            pl.BlockSpec((1, H), lambda i: (0, 0)),
            pl.BlockSpec((H, H), lambda i: (0, 0)),
            pl.BlockSpec((1, H), lambda i: (0, 0)),
            pl.BlockSpec((H, H), lambda i: (0, 0)),
        ],
        out_specs=[pl.BlockSpec((RB, H), lambda i: (i, 0)),
                   pl.BlockSpec((RB, H), lambda i: (i, 0))],
        out_shape=[jax.ShapeDtypeStruct((NP, H), F32),
                   jax.ShapeDtypeStruct((NP, H), F32)],
    )(part, inb, onb, bg, wf, bf, wgn)


def _bitonic_lanes(X):
    R, n = X.shape
    li = lax.broadcasted_iota(I32, (R, n), 1)
    k = 2
    while k <= n:
        j = k // 2
        while j >= 1:
            bitj0 = (li & j) == 0
            asc = (li & k) == 0
            keepmin = bitj0 == asc
            left = jnp.concatenate([X[:, j:], X[:, :j]], axis=1)
            right = jnp.concatenate([X[:, n - j:], X[:, :n - j]], axis=1)
            xp = jnp.where(bitj0, left, right)
            X = jnp.where(keepmin, jnp.minimum(X, xp), jnp.maximum(X, xp))
            j //= 2
        k *= 2
    return X


def _tc_pre4(part4, inb, bg4, wf4, bf4):
    def body(p_ref, i_ref, bg_ref, wf_ref, bf_ref, a4_ref):
        inorm = jnp.power(jnp.maximum(i_ref[...], 1.0), -0.5)
        g4 = (p_ref[0] + p_ref[1]) * inorm + bg_ref[...]
        a4 = jnp.sum(jnp.maximum(g4, 0.0) * wf_ref[...], axis=1,
                     keepdims=True) + bf_ref[...]
        a4_ref[...] = jnp.broadcast_to(jnp.tanh(a4), (RB, H))

    blk = pl.BlockSpec((RB, H), lambda i: (i, 0))
    cst = lambda shape: pl.BlockSpec(shape, lambda i: tuple(0 for _ in shape))
    return pl.pallas_call(
        body,
        grid=(NP // RB,),
        in_specs=[
            pl.BlockSpec((2, RB, H), lambda i: (0, i, 0)),
            blk,
            cst((1, H)), cst((1, H)), cst((1, 1)),
        ],
        out_specs=blk,
        out_shape=jax.ShapeDtypeStruct((NP, H), F32),
    )(part4, inb, bg4, wf4, bf4)


def _tc_rowmax(h1, h2, h3, h4b):
    def body(h1_ref, h2_ref, h3_ref, h4_ref, m_ref):
        i = pl.program_id(0)
        m = jnp.maximum(
            jnp.maximum(jnp.max(h1_ref[...], axis=1, keepdims=True),
                        jnp.max(h2_ref[...], axis=1, keepdims=True)),
            jnp.maximum(jnp.max(h3_ref[...], axis=1, keepdims=True),
                        h4_ref[:, 0:1]))
        rid = i * RB + lax.broadcasted_iota(I32, (RB, 1), 0)
        m = jnp.where(rid < N, m, NEG_INF)
        m_ref[...] = jnp.broadcast_to(m, (RB, H))

    blk = pl.BlockSpec((RB, H), lambda i: (i, 0))
    return pl.pallas_call(
        body,
        grid=(NP // RB,),
        in_specs=[blk, blk, blk, blk],
        out_specs=blk,
        out_shape=jax.ShapeDtypeStruct((NP, H), F32),
    )(h1, h2, h3, h4b)


def _tc_topk(mb):
    def body(m_ref, oh_ref):
        m = m_ref[:, 0:1]
        rid = lax.broadcasted_iota(I32, (NP, 1), 0)
        row_iota = lax.broadcasted_iota(I32, (1, NP), 1)
        sel_iota = lax.broadcasted_iota(I32, (32, 1), 0)

        def topk_body(j, carry):
            mc, oh = carry
            vmax = jnp.max(mc)
            idx = jnp.min(jnp.where(mc == vmax, rid, NP))
            oh = oh + ((sel_iota == j).astype(F32) *
                       (row_iota == idx).astype(F32))
            mc = jnp.where(rid == idx, NEG_INF, mc)
            return (mc, oh)

        _, oh = lax.fori_loop(0, K, topk_body,
                              (m, jnp.zeros((32, NP), F32)))
        oh_ref[...] = oh

    return pl.pallas_call(
        body,
        out_shape=jax.ShapeDtypeStruct((32, NP), F32),
    )(mb)


def _tc_gather(oh, h1, h2, h3, h4b):
    def body(oh_ref, h1_ref, h2_ref, h3_ref, h4_ref, out_ref):
        i = pl.program_id(0)
        dn = (((1,), (0,)), ((), ()))
        hp = dict(preferred_element_type=F32,
                  precision=lax.Precision.HIGHEST)
        ohb = oh_ref[...]
        p1 = lax.dot_general(ohb, h1_ref[...], dn, **hp)
        p2 = lax.dot_general(ohb, h2_ref[...], dn, **hp)
        p3 = lax.dot_general(ohb, h3_ref[...], dn, **hp)
        p4 = lax.dot_general(ohb, h4_ref[:, 0:1], dn, **hp)
        part = jnp.concatenate([p1, p2, p3, p4, jnp.zeros((32, 127), F32)],
                               axis=1)

        @pl.when(i == 0)
        def _():
            out_ref[...] = jnp.zeros((32, 512), F32)

        out_ref[...] += part

    blk = pl.BlockSpec((RB, H), lambda i: (i, 0))
    return pl.pallas_call(
        body,
        grid=(NP // RB,),
        in_specs=[pl.BlockSpec((32, RB), lambda i: (0, i)),
                  blk, blk, blk, blk],
        out_specs=pl.BlockSpec((32, 512), lambda i: (0, 0)),
        out_shape=jax.ShapeDtypeStruct((32, 512), F32),
        compiler_params=pltpu.CompilerParams(
            dimension_semantics=("arbitrary",)),
    )(oh, h1, h2, h3, h4b)


def _tc_readout(pooled, w1r, b1, se, so, st, w2r, b2, l1r, l1b, l2w, l2b):
    def body(pool_ref, w1_ref, b1_ref, se_ref, so_ref, st_ref, w2_ref,
             b2_ref, l1_ref, l1b_ref, l2_ref, l2b_ref, out_ref):
        lane = lax.broadcasted_iota(I32, (32, 512), 1)
        x32 = jnp.where(lane < D_LAT, pool_ref[...], jnp.inf)
        xs = _bitonic_lanes(x32)[0:K, 0:D_LAT]
        y1 = jnp.maximum(
            lax.dot_general(xs, w1_ref[...], (((1,), (0,)), ((), ())),
                            preferred_element_type=F32) + b1_ref[...], 0.0)
        hp = dict(preferred_element_type=F32,
                  precision=lax.Precision.HIGHEST)
        z = jnp.maximum(
            lax.dot_general(se_ref[...], y1, (((1,), (0,)), ((), ())), **hp),
            lax.dot_general(so_ref[...], y1, (((1,), (0,)), ((), ())), **hp))
        wins = jnp.concatenate(
            [lax.dot_general(st_ref[t], z, (((1,), (0,)), ((), ())), **hp)
             for t in range(5)],
            axis=1)
        y2 = jnp.maximum(
            lax.dot_general(wins, w2_ref[...], (((1,), (0,)), ((), ())),
                            preferred_element_type=F32) + b2_ref[...], 0.0)
        accum = l1b_ref[...]
        for p in range(11):
            accum = accum + lax.dot_general(
                y2[p:p + 1, :], l1_ref[p], (((1,), (0,)), ((), ())),
                preferred_element_type=F32)
        x = jnp.maximum(accum, 0.0)
        out_ref[...] = jnp.sum(x * l2_ref[...], axis=1,
                               keepdims=True) + l2b_ref[...]

    return pl.pallas_call(
        body,
        out_shape=jax.ShapeDtypeStruct((1, 1), F32),
    )(pooled, w1r, b1, se, so, st, w2r, b2, l1r, l1b, l2w, l2b)


def kernel(params, z, edge_index):
    src = edge_index[0].astype(I32)
    dst = edge_index[1].astype(I32)
    pad = N + (jnp.arange(E_PAD - E, dtype=I32) % NDUM)
    srcp = jnp.concatenate([src, pad])
    dstp = jnp.concatenate([dst, pad])
    z_pad = jnp.concatenate([z.astype(I32), jnp.zeros((NP - N,), I32)])

    convs = params['convs']
    onb, inb = _sc_deg_norm(srcp, dstp)
    h0 = _sc_embed(params['z_table'], z_pad)

    t = _tc_first(h0, onb, convs[0]['Wg'])
    hs = []
    for i in range(3):
        part = _sc_agg(t, srcp, dstp)
        h, t = _tc_combine(part, inb, onb,
                           convs[i]['bg'].reshape(1, H),
                           convs[i]['Wf'],
                           convs[i]['bf'].reshape(1, H),
                           convs[i + 1]['Wg'])
        hs.append(h)
    part4 = _sc_agg(t, srcp, dstp)
    h4b = _tc_pre4(part4, inb,
                   convs[3]['bg'].reshape(1, H),
                   convs[3]['Wf'].reshape(1, H),
                   convs[3]['bf'].reshape(1, 1))
    mb = _tc_rowmax(hs[0], hs[1], hs[2], h4b)

    se = np.zeros((15, K), np.float32)
    so = np.zeros((15, K), np.float32)
    for kk in range(15):
        se[kk, 2 * kk] = 1.0
        so[kk, 2 * kk + 1] = 1.0
    st = np.zeros((5, 11, 15), np.float32)
    for t_ in range(5):
        for p_ in range(11):
            st[t_, p_, p_ + t_] = 1.0

    w1r = params['conv1_W'][:, 0, :].T
    w2r = jnp.transpose(params['conv2_W'], (2, 1, 0)).reshape(80, 32)
    l1r = params['lin1_W'].reshape(32, 11, H).transpose(1, 0, 2)

    oh = _tc_topk(mb)
    pooled = _tc_gather(oh, hs[0], hs[1], hs[2], h4b)
    out = _tc_readout(
        pooled,
        w1r,
        params['conv1_b'].reshape(1, 16),
        jnp.asarray(se), jnp.asarray(so), jnp.asarray(st),
        w2r,
        params['conv2_b'].reshape(1, 32),
        l1r,
        params['lin1_b'].reshape(1, H),
        params['lin2_W'].reshape(1, H),
        params['lin2_b'].reshape(1, 1),
    )
    return out
